# Initial kernel scaffold; baseline (speedup 1.0000x reference)
import jax
import jax.numpy as jnp
from jax.experimental import pallas as pl


def kernel(tokenized_prompts, token_embedding_table):
    raise NotImplementedError("write your pallas kernel here")


# speedup vs baseline: 1.0490x; 1.0490x over previous
import functools

import jax
import jax.numpy as jnp
from jax import lax
from jax.experimental import pallas as pl
from jax.experimental.pallas import tpu as pltpu
from jax.experimental.pallas import tpu_sc as plsc

_NC = 2
_NS = 16
_NW = _NC * _NS
_C = 112


def _sc_gather(table, idx3d, n_chunks, out_rows, d):
    mesh = plsc.VectorSubcoreMesh(core_axis_name="c", subcore_axis_name="s")

    @functools.partial(
        pl.kernel,
        out_type=jax.ShapeDtypeStruct((out_rows, d), table.dtype),
        mesh=mesh,
        scratch_types=[
            pltpu.VMEM((n_chunks, _C), jnp.int32),
            pltpu.VMEM((_C, d), table.dtype),
            pltpu.SemaphoreType.DMA,
        ],
    )
    def k(table_hbm, idx_hbm, out_hbm, idx_v, rows_v, sem):
        wid = lax.axis_index("s") * _NC + lax.axis_index("c")
        pltpu.sync_copy(idx_hbm.at[wid], idx_v)

        @pl.loop(0, n_chunks)
        def _(j):
            pltpu.async_copy(table_hbm.at[idx_v.at[j]], rows_v, sem).wait()
            pltpu.sync_copy(
                rows_v, out_hbm.at[pl.ds((wid * n_chunks + j) * _C, _C)]
            )

    return k(table, idx3d)


def kernel(tokenized_prompts, token_embedding_table):
    n, ctx = tokenized_prompts.shape
    _, d = token_embedding_table.shape
    b = n * ctx
    sweep = _C * _NW
    b_pad = ((b + sweep - 1) // sweep) * sweep
    n_chunks = b_pad // sweep
    flat = tokenized_prompts.reshape(-1)
    flat = jnp.pad(flat, (0, b_pad - b))
    idx3d = flat.reshape(_NW, n_chunks, _C)
    out = _sc_gather(token_embedding_table, idx3d, n_chunks, b_pad, d)
    prompts = out[:b].reshape(n, ctx, d)
    return (prompts, tokenized_prompts)
